# baseline (speedup 1.0000x reference)
import functools

import jax
import jax.numpy as jnp
from jax import lax
from jax.experimental import pallas as pl
from jax.experimental.pallas import tpu as pltpu
from jax.experimental.pallas import tpu_sc as plsc

_B = 4096
_L = 200
_D = 32
_V = 1000000
_NL = 10
_NC = 2
_NS = 16
_NW = _NC * _NS
_BPW = _B // _NW
_HALF = _D // 16
_NSPLIT = _BPW // 32


def _make_sc_pool():
    mesh = plsc.VectorSubcoreMesh(core_axis_name="c", subcore_axis_name="s")

    @functools.partial(
        pl.kernel,
        out_type=jax.ShapeDtypeStruct((_B, _D), jnp.float32),
        mesh=mesh,
        scratch_types=[
            [pltpu.VMEM((_L, 32), jnp.float32) for _ in range(_NSPLIT)],
            [pltpu.VMEM((_L, 32), jnp.int32) for _ in range(_NSPLIT)],
            pltpu.VMEM((_BPW, _D), jnp.float32),
            pltpu.SemaphoreType.DMA,
        ],
        compiler_params=pltpu.CompilerParams(
            use_tc_tiling_on_sc=False, needs_layout_passes=False
        ),
    )
    def sc_pool(tab_hbm, x0, x1, x2, x3, out_hbm, xf_vs, idx_vs, acc, sem):
        wid = lax.axis_index("s") * _NC + lax.axis_index("c")
        for xk, xf_v in zip((x0, x1, x2, x3), xf_vs):
            pltpu.sync_copy(xk.at[pl.ds(wid * _L, _L)], xf_v)

        @pl.loop(0, _L)
        def _toint(l):
            for k in range(_NSPLIT):
                for h in range(2):
                    v = xf_vs[k][l, pl.ds(16 * h, 16)]
                    idx_vs[k][l, pl.ds(16 * h, 16)] = plsc.bitcast(v, jnp.int32)

        zero = jnp.zeros((16,), jnp.float32)
        for r in range(_BPW):
            for h in range(_HALF):
                acc[r, pl.ds(16 * h, 16)] = zero

        @pl.loop(0, _L)
        def _fire(l):
            for k in range(_NSPLIT):
                pltpu.async_copy(
                    tab_hbm.at[idx_vs[k].at[l]],
                    acc.at[pl.ds(32 * k, 32)],
                    sem,
                    add=True,
                )

        @pl.loop(0, _L)
        def _drain(l):
            for k in range(_NSPLIT):
                pltpu.make_async_copy(
                    tab_hbm.at[idx_vs[k].at[0]], acc.at[pl.ds(32 * k, 32)], sem
                ).wait()

        pltpu.sync_copy(acc, out_hbm.at[pl.ds(wid * _BPW, _BPW)])

    return sc_pool


_sc_pool = _make_sc_pool()


def _xt_body(x_ref, o0_ref, o1_ref, o2_ref, o3_ref):
    xt = jax.lax.bitcast_convert_type(jnp.transpose(x_ref[...], (1, 0)), jnp.float32)
    o0_ref[...] = xt[:, 0:32]
    o1_ref[...] = xt[:, 32:64]
    o2_ref[...] = xt[:, 64:96]
    o3_ref[...] = xt[:, 96:128]


def _xt(x):
    out = jax.ShapeDtypeStruct((_NW * _L, 32), jnp.float32)
    spec = pl.BlockSpec((_L, 32), lambda w: (w, 0))
    return pl.pallas_call(
        _xt_body,
        grid=(_NW,),
        in_specs=[pl.BlockSpec((_BPW, _L), lambda w: (w, 0))],
        out_specs=[spec] * _NSPLIT,
        out_shape=[out] * _NSPLIT,
    )(x)


def _head_body(doc_ref, w_ref, b_ref, out_ref):
    out_ref[...] = (
        jnp.dot(doc_ref[...], w_ref[...], preferred_element_type=jnp.float32)
        + b_ref[...]
    )


def _head(doc, W, b2):
    return pl.pallas_call(
        _head_body,
        out_shape=jax.ShapeDtypeStruct((_B, _NL), jnp.float32),
    )(doc, W, b2)


def kernel(x, m, table, W, b):
    del m
    x0, x1, x2, x3 = _xt(x.astype(jnp.int32))
    doc = _sc_pool(table, x0, x1, x2, x3)
    return _head(doc, W, b.reshape(1, _NL))


# speedup vs baseline: 1.0133x; 1.0133x over previous
import functools

import jax
import jax.numpy as jnp
from jax import lax
from jax.experimental import pallas as pl
from jax.experimental.pallas import tpu as pltpu
from jax.experimental.pallas import tpu_sc as plsc

_B = 4096
_L = 200
_D = 32
_V = 1000000
_NL = 10
_NC = 2
_NS = 16
_NW = _NC * _NS
_BPW = _B // _NW
_HALF = _D // 16


def _make_sc_pool():
    mesh = plsc.VectorSubcoreMesh(core_axis_name="c", subcore_axis_name="s")

    @functools.partial(
        pl.kernel,
        out_type=jax.ShapeDtypeStruct((_B, _D), jnp.float32),
        mesh=mesh,
        scratch_types=[
            pltpu.VMEM((_L, _BPW), jnp.int32),
            pltpu.VMEM((_BPW, _D), jnp.float32),
            pltpu.SemaphoreType.DMA,
        ],
        compiler_params=pltpu.CompilerParams(
            use_tc_tiling_on_sc=False, needs_layout_passes=False
        ),
    )
    def sc_pool(xt_hbm, tab_hbm, out_hbm, idx_v, acc, sem):
        wid = lax.axis_index("s") * _NC + lax.axis_index("c")
        pltpu.sync_copy(xt_hbm.at[pl.ds(wid * _L, _L)], idx_v)

        zero = jnp.zeros((16,), jnp.float32)
        for r in range(_BPW):
            for h in range(_HALF):
                acc[r, pl.ds(16 * h, 16)] = zero

        @pl.loop(0, _L)
        def _fire(l):
            pltpu.async_copy(tab_hbm.at[idx_v.at[l]], acc, sem, add=True)

        @pl.loop(0, _L)
        def _drain(l):
            pltpu.make_async_copy(tab_hbm.at[idx_v.at[0]], acc, sem).wait()

        pltpu.sync_copy(acc, out_hbm.at[pl.ds(wid * _BPW, _BPW)])

    return sc_pool


_sc_pool = _make_sc_pool()


def _xt_body(x_ref, o_ref):
    o_ref[...] = jnp.transpose(x_ref[...], (1, 0))


def _xt(x):
    return pl.pallas_call(
        _xt_body,
        grid=(_NW,),
        in_specs=[pl.BlockSpec((_BPW, _L), lambda w: (w, 0))],
        out_specs=pl.BlockSpec((_L, _BPW), lambda w: (w, 0)),
        out_shape=jax.ShapeDtypeStruct((_NW * _L, _BPW), jnp.int32),
    )(x)


def _head_body(doc_ref, w_ref, b_ref, out_ref):
    out_ref[...] = (
        jnp.dot(doc_ref[...], w_ref[...], preferred_element_type=jnp.float32)
        + b_ref[...]
    )


def _head(doc, W, b2):
    return pl.pallas_call(
        _head_body,
        out_shape=jax.ShapeDtypeStruct((_B, _NL), jnp.float32),
    )(doc, W, b2)


def kernel(x, m, table, W, b):
    del m
    xt = _xt(x.astype(jnp.int32))
    doc = _sc_pool(xt, table)
    return _head(doc, W, b.reshape(1, _NL))
